# baseline (speedup 1.0000x reference)
import jax
import jax.numpy as jnp
from jax import lax
from jax.experimental import pallas as pl
from jax.experimental.pallas import tpu as pltpu
from jax.experimental.pallas import tpu_sc as plsc

N_NODES = 10000
N_EDGES = 320000
D_FEAT = 128
D_EDGE = 16
LATENT = 128

_NC = 2
_NS = 16
_NW = _NC * _NS

_CHUNK = 128
_N_CHUNKS = N_EDGES // _CHUNK
_MAX_I = -(-_N_CHUNKS // _NW)

_RING = 4
_LOOP_HI = -(-(_MAX_I + 2) // _RING) * _RING


def _preproj_body(s_ref, r_ref, ws_ref, wr_ref, ps_ref, pr_ref):
    ps_ref[...] = jnp.dot(s_ref[...], ws_ref[...],
                          preferred_element_type=jnp.float32)
    pr_ref[...] = jnp.dot(r_ref[...], wr_ref[...],
                          preferred_element_type=jnp.float32)


def _preproject(s, r, w0s, w0r):
    return pl.pallas_call(
        _preproj_body,
        out_shape=(
            jax.ShapeDtypeStruct((N_NODES, D_FEAT), jnp.float32),
            jax.ShapeDtypeStruct((N_NODES, D_FEAT), jnp.float32),
        ),
    )(s, r, w0s, w0r)


def _gather_body(ps_hbm, pr_hbm, idx_hbm, g_hbm,
                 idx0, idx1, idx2, idx3, buf0, buf1, buf2, buf3,
                 gsem0, gsem1, gsem2, gsem3,
                 asem0, asem1, asem2, asem3,
                 wsem0, wsem1, wsem2, wsem3,
                 isem0, isem1, isem2, isem3):
    idx_v = (idx0, idx1, idx2, idx3)
    buf = (buf0, buf1, buf2, buf3)
    gsem = (gsem0, gsem1, gsem2, gsem3)
    asem = (asem0, asem1, asem2, asem3)
    wsem = (wsem0, wsem1, wsem2, wsem3)
    isem = (isem0, isem1, isem2, isem3)
    wid = lax.axis_index("s") * _NC + lax.axis_index("c")

    def chunk_of(i):
        return wid + i * _NW

    def issue_i(i, s):
        j = chunk_of(i)

        @pl.when(jnp.logical_and(i >= 0, j < _N_CHUNKS))
        def _():
            pltpu.async_copy(idx_hbm.at[j], idx_v[s], isem[s])

    def issue_a(i, s):
        j = chunk_of(i)

        @pl.when(jnp.logical_and(i >= 0, j < _N_CHUNKS))
        def _():
            @pl.when(i >= _RING)
            def _():
                pltpu.make_async_copy(
                    buf[s], g_hbm.at[pl.ds(0, _CHUNK)], wsem[s]).wait()
            pltpu.make_async_copy(idx_hbm.at[0], idx_v[s], isem[s]).wait()
            pltpu.async_copy(
                ps_hbm.at[idx_v[s].at[pl.ds(0, _CHUNK)]], buf[s], gsem[s])

    def issue_c1(i, s):
        j = chunk_of(i)

        @pl.when(jnp.logical_and(i >= 0, j < _N_CHUNKS))
        def _():
            pltpu.make_async_copy(
                ps_hbm.at[pl.ds(0, _CHUNK)], buf[s], gsem[s]).wait()
            pltpu.async_copy(
                pr_hbm.at[idx_v[s].at[pl.ds(_CHUNK, _CHUNK)]], buf[s],
                asem[s], add=True)

    def issue_c2(i, s):
        j = chunk_of(i)

        @pl.when(jnp.logical_and(i >= 0, j < _N_CHUNKS))
        def _():
            pltpu.make_async_copy(
                pr_hbm.at[pl.ds(0, _CHUNK)], buf[s], asem[s]).wait()
            pltpu.async_copy(buf[s], g_hbm.at[pl.ds(j * _CHUNK, _CHUNK)],
                             wsem[s])

    issue_i(0, 0)
    issue_i(1, 1)

    @pl.loop(0, _LOOP_HI, step=_RING)
    def _step(i0):
        for b in range(_RING):
            i = i0 + b
            issue_a(i, b)
            issue_c1(i - 1, (b - 1) % _RING)
            issue_c2(i - 2, (b - 2) % _RING)
            issue_i(i + 2, (b + 2) % _RING)

    for s in range(_RING):
        pltpu.make_async_copy(
            buf[s], g_hbm.at[pl.ds(0, _CHUNK)], wsem[s]).wait()


def _sc_gather(ps, pr, idx2d):
    mesh = plsc.VectorSubcoreMesh(core_axis_name="c", subcore_axis_name="s")
    return pl.kernel(
        _gather_body,
        out_type=jax.ShapeDtypeStruct((N_EDGES, D_FEAT), jnp.float32),
        mesh=mesh,
        scratch_types=[
            pltpu.VMEM((2 * _CHUNK,), jnp.int32),
            pltpu.VMEM((2 * _CHUNK,), jnp.int32),
            pltpu.VMEM((2 * _CHUNK,), jnp.int32),
            pltpu.VMEM((2 * _CHUNK,), jnp.int32),
            pltpu.VMEM((_CHUNK, D_FEAT), jnp.float32),
            pltpu.VMEM((_CHUNK, D_FEAT), jnp.float32),
            pltpu.VMEM((_CHUNK, D_FEAT), jnp.float32),
            pltpu.VMEM((_CHUNK, D_FEAT), jnp.float32),
            pltpu.SemaphoreType.DMA,
            pltpu.SemaphoreType.DMA,
            pltpu.SemaphoreType.DMA,
            pltpu.SemaphoreType.DMA,
            pltpu.SemaphoreType.DMA,
            pltpu.SemaphoreType.DMA,
            pltpu.SemaphoreType.DMA,
            pltpu.SemaphoreType.DMA,
            pltpu.SemaphoreType.DMA,
            pltpu.SemaphoreType.DMA,
            pltpu.SemaphoreType.DMA,
            pltpu.SemaphoreType.DMA,
            pltpu.SemaphoreType.DMA,
            pltpu.SemaphoreType.DMA,
            pltpu.SemaphoreType.DMA,
            pltpu.SemaphoreType.DMA,
        ],
    )(ps, pr, idx2d)


_MLP_BLOCK = 16000


def _mlp_body(g_ref, e_ref, w0e_ref, b0_ref, w1_ref, b1_ref,
              w2_ref, b2_ref, out_ref):
    h0 = (g_ref[...] + b0_ref[...]
          + jnp.dot(e_ref[...], w0e_ref[...],
                    preferred_element_type=jnp.float32))
    h0 = jnp.maximum(h0, 0.0).astype(jnp.bfloat16)
    h1 = jnp.dot(h0, w1_ref[...], preferred_element_type=jnp.float32)
    h1 = jnp.maximum(h1 + b1_ref[...], 0.0).astype(jnp.bfloat16)
    out_ref[...] = (jnp.dot(h1, w2_ref[...],
                            preferred_element_type=jnp.float32)
                    + b2_ref[...])


def _mlp(g, e, w0e, b0, w1, b1, w2, b2):
    n_blocks = N_EDGES // _MLP_BLOCK
    row_spec = lambda width: pl.BlockSpec((_MLP_BLOCK, width),
                                          lambda i: (i, 0))
    full = lambda shape: pl.BlockSpec(shape, lambda i: (0, 0))
    return pl.pallas_call(
        _mlp_body,
        grid=(n_blocks,),
        in_specs=[
            row_spec(LATENT),
            row_spec(D_EDGE),
            full((D_EDGE, LATENT)),
            full((1, LATENT)),
            full((LATENT, LATENT)),
            full((1, LATENT)),
            full((LATENT, LATENT)),
            full((1, LATENT)),
        ],
        out_specs=row_spec(LATENT),
        out_shape=jax.ShapeDtypeStruct((N_EDGES, LATENT), jnp.float32),
    )(g, e, w0e, b0, w1, b1, w2, b2)


def kernel(sender_features, receiver_features, edge_features, senders,
           receivers, W0, b0, W1, b1, W2, b2):
    w0s = W0[:D_FEAT]
    w0r = W0[D_FEAT:2 * D_FEAT]
    w0e = W0[2 * D_FEAT:]
    ps, pr = _preproject(sender_features, receiver_features, w0s, w0r)
    snd2d = senders.astype(jnp.int32).reshape(_N_CHUNKS, _CHUNK)
    rcv2d = receivers.astype(jnp.int32).reshape(_N_CHUNKS, _CHUNK)
    idx2d = jnp.concatenate([snd2d, rcv2d], axis=1)
    g = _sc_gather(ps, pr, idx2d)
    return _mlp(g, edge_features,
                w0e, b0.reshape(1, LATENT),
                W1.astype(jnp.bfloat16), b1.reshape(1, LATENT),
                W2.astype(jnp.bfloat16), b2.reshape(1, LATENT))


# speedup vs baseline: 1.0021x; 1.0021x over previous
import jax
import jax.numpy as jnp
from jax import lax
from jax.experimental import pallas as pl
from jax.experimental.pallas import tpu as pltpu
from jax.experimental.pallas import tpu_sc as plsc

N_NODES = 10000
N_EDGES = 320000
D_FEAT = 128
D_EDGE = 16
LATENT = 128

_NC = 2
_NS = 16
_NW = _NC * _NS

_CHUNK = 128
_N_CHUNKS = N_EDGES // _CHUNK
_MAX_I = -(-_N_CHUNKS // _NW)

_RING = 4
_LOOP_HI = -(-(_MAX_I + 2) // _RING) * _RING


def _preproj_body(s_ref, r_ref, ws_ref, wr_ref, ps_ref, pr_ref):
    ps_ref[...] = jnp.dot(s_ref[...], ws_ref[...],
                          preferred_element_type=jnp.float32)
    pr_ref[...] = jnp.dot(r_ref[...], wr_ref[...],
                          preferred_element_type=jnp.float32)


def _preproject(s, r, w0s, w0r):
    return pl.pallas_call(
        _preproj_body,
        out_shape=(
            jax.ShapeDtypeStruct((N_NODES, D_FEAT), jnp.float32),
            jax.ShapeDtypeStruct((N_NODES, D_FEAT), jnp.float32),
        ),
    )(s, r, w0s, w0r)


def _gather_body(ps_hbm, pr_hbm, idx_hbm, g_hbm,
                 idx0, idx1, idx2, idx3, buf0, buf1, buf2, buf3,
                 gsem0, gsem1, gsem2, gsem3,
                 asem0, asem1, asem2, asem3,
                 wsem0, wsem1, wsem2, wsem3,
                 isem0, isem1, isem2, isem3):
    idx_v = (idx0, idx1, idx2, idx3)
    buf = (buf0, buf1, buf2, buf3)
    gsem = (gsem0, gsem1, gsem2, gsem3)
    asem = (asem0, asem1, asem2, asem3)
    wsem = (wsem0, wsem1, wsem2, wsem3)
    isem = (isem0, isem1, isem2, isem3)
    wid = lax.axis_index("s") * _NC + lax.axis_index("c")

    def chunk_of(i):
        return wid + i * _NW

    def issue_i(i, s):
        j = chunk_of(i)

        @pl.when(jnp.logical_and(i >= 0, j < _N_CHUNKS))
        def _():
            pltpu.async_copy(idx_hbm.at[j], idx_v[s], isem[s])

    def issue_a(i, s):
        j = chunk_of(i)

        @pl.when(jnp.logical_and(i >= 0, j < _N_CHUNKS))
        def _():
            @pl.when(i >= _RING)
            def _():
                pltpu.make_async_copy(
                    buf[s], g_hbm.at[pl.ds(0, _CHUNK)], wsem[s]).wait()
            pltpu.make_async_copy(idx_hbm.at[0], idx_v[s], isem[s]).wait()
            pltpu.async_copy(
                ps_hbm.at[idx_v[s].at[pl.ds(0, _CHUNK)]], buf[s], gsem[s])

    def issue_c1(i, s):
        j = chunk_of(i)

        @pl.when(jnp.logical_and(i >= 0, j < _N_CHUNKS))
        def _():
            pltpu.make_async_copy(
                ps_hbm.at[pl.ds(0, _CHUNK)], buf[s], gsem[s]).wait()
            pltpu.async_copy(
                pr_hbm.at[idx_v[s].at[pl.ds(_CHUNK, _CHUNK)]], buf[s],
                asem[s], add=True)

    def issue_c2(i, s):
        j = chunk_of(i)

        @pl.when(jnp.logical_and(i >= 0, j < _N_CHUNKS))
        def _():
            pltpu.make_async_copy(
                pr_hbm.at[pl.ds(0, _CHUNK)], buf[s], asem[s]).wait()
            pltpu.async_copy(buf[s], g_hbm.at[pl.ds(j * _CHUNK, _CHUNK)],
                             wsem[s])

    issue_i(0, 0)
    issue_i(1, 1)

    @pl.loop(0, _LOOP_HI, step=_RING)
    def _step(i0):
        for b in range(_RING):
            i = i0 + b
            issue_a(i, b)
            issue_c1(i - 1, (b - 1) % _RING)
            issue_c2(i - 2, (b - 2) % _RING)
            issue_i(i + 2, (b + 2) % _RING)

    for s in range(_RING):
        pltpu.make_async_copy(
            buf[s], g_hbm.at[pl.ds(0, _CHUNK)], wsem[s]).wait()


def _sc_gather(ps, pr, idx2d):
    mesh = plsc.VectorSubcoreMesh(core_axis_name="c", subcore_axis_name="s")
    return pl.kernel(
        _gather_body,
        out_type=jax.ShapeDtypeStruct((N_EDGES, D_FEAT), jnp.float32),
        mesh=mesh,
        scratch_types=[
            pltpu.VMEM((2 * _CHUNK,), jnp.int32),
            pltpu.VMEM((2 * _CHUNK,), jnp.int32),
            pltpu.VMEM((2 * _CHUNK,), jnp.int32),
            pltpu.VMEM((2 * _CHUNK,), jnp.int32),
            pltpu.VMEM((_CHUNK, D_FEAT), jnp.float32),
            pltpu.VMEM((_CHUNK, D_FEAT), jnp.float32),
            pltpu.VMEM((_CHUNK, D_FEAT), jnp.float32),
            pltpu.VMEM((_CHUNK, D_FEAT), jnp.float32),
            pltpu.SemaphoreType.DMA,
            pltpu.SemaphoreType.DMA,
            pltpu.SemaphoreType.DMA,
            pltpu.SemaphoreType.DMA,
            pltpu.SemaphoreType.DMA,
            pltpu.SemaphoreType.DMA,
            pltpu.SemaphoreType.DMA,
            pltpu.SemaphoreType.DMA,
            pltpu.SemaphoreType.DMA,
            pltpu.SemaphoreType.DMA,
            pltpu.SemaphoreType.DMA,
            pltpu.SemaphoreType.DMA,
            pltpu.SemaphoreType.DMA,
            pltpu.SemaphoreType.DMA,
            pltpu.SemaphoreType.DMA,
            pltpu.SemaphoreType.DMA,
        ],
    )(ps, pr, idx2d)


_MLP_BLOCK = 16000


def _mlp_body(g_ref, e_ref, w0e_ref, b0_ref, w1_ref, b1_ref,
              w2_ref, b2_ref, out_ref):
    h0 = (g_ref[...] + b0_ref[...]
          + jnp.dot(e_ref[...], w0e_ref[...],
                    preferred_element_type=jnp.float32))
    h0 = jnp.maximum(h0, 0.0)
    h1 = jnp.dot(h0, w1_ref[...], preferred_element_type=jnp.float32)
    h1 = jnp.maximum(h1 + b1_ref[...], 0.0)
    out_ref[...] = (jnp.dot(h1, w2_ref[...],
                            preferred_element_type=jnp.float32)
                    + b2_ref[...])


def _mlp(g, e, w0e, b0, w1, b1, w2, b2):
    n_blocks = N_EDGES // _MLP_BLOCK
    row_spec = lambda width: pl.BlockSpec((_MLP_BLOCK, width),
                                          lambda i: (i, 0))
    full = lambda shape: pl.BlockSpec(shape, lambda i: (0, 0))
    return pl.pallas_call(
        _mlp_body,
        grid=(n_blocks,),
        in_specs=[
            row_spec(LATENT),
            row_spec(D_EDGE),
            full((D_EDGE, LATENT)),
            full((1, LATENT)),
            full((LATENT, LATENT)),
            full((1, LATENT)),
            full((LATENT, LATENT)),
            full((1, LATENT)),
        ],
        out_specs=row_spec(LATENT),
        out_shape=jax.ShapeDtypeStruct((N_EDGES, LATENT), jnp.float32),
    )(g, e, w0e, b0, w1, b1, w2, b2)


def kernel(sender_features, receiver_features, edge_features, senders,
           receivers, W0, b0, W1, b1, W2, b2):
    w0s = W0[:D_FEAT]
    w0r = W0[D_FEAT:2 * D_FEAT]
    w0e = W0[2 * D_FEAT:]
    ps, pr = _preproject(sender_features, receiver_features, w0s, w0r)
    snd2d = senders.astype(jnp.int32).reshape(_N_CHUNKS, _CHUNK)
    rcv2d = receivers.astype(jnp.int32).reshape(_N_CHUNKS, _CHUNK)
    idx2d = jnp.concatenate([snd2d, rcv2d], axis=1)
    g = _sc_gather(ps, pr, idx2d)
    return _mlp(g, edge_features,
                w0e, b0.reshape(1, LATENT),
                W1, b1.reshape(1, LATENT),
                W2, b2.reshape(1, LATENT))
